# speedup vs baseline: 8.9893x; 8.9893x over previous
import functools

import jax
import jax.numpy as jnp
from jax import lax
from jax.experimental import pallas as pl
from jax.experimental.pallas import tpu as pltpu
from jax.experimental.pallas import tpu_sc as plsc

D = 128
NC = 2
NS = 16
NW = NC * NS

CHUNK_K = 4
CHUNK = CHUNK_K * 128


@functools.lru_cache(maxsize=None)
def _make_kernel(B):
    assert B % (NW * CHUNK) == 0
    rows_per_w = B // (NW * 128)
    n_iters = rows_per_w // CHUNK_K

    mesh = plsc.VectorSubcoreMesh(core_axis_name="c", subcore_axis_name="s")

    @functools.partial(
        pl.kernel,
        out_type=jax.ShapeDtypeStruct((B, D), jnp.float32),
        mesh=mesh,
        scratch_types=[
            pltpu.VMEM((CHUNK_K, 128), jnp.int32),
            pltpu.VMEM((CHUNK, D), jnp.float32),
            pltpu.SemaphoreType.DMA,
        ],
    )
    def gather_kernel(ids_hbm, pe_hbm, out_hbm, idx_v, rows_v, sem):
        wid = lax.axis_index("s") * NC + lax.axis_index("c")
        row_base = wid * rows_per_w

        def body(i, carry):
            off = row_base + i * CHUNK_K
            pltpu.sync_copy(ids_hbm.at[pl.ds(off, CHUNK_K)], idx_v)
            copies = []
            for j in range(CHUNK_K):
                copies.append(
                    pltpu.async_copy(
                        pe_hbm.at[idx_v.at[j]],
                        rows_v.at[pl.ds(j * 128, 128)],
                        sem,
                    )
                )
            for cp in copies:
                cp.wait()
            pltpu.sync_copy(rows_v, out_hbm.at[pl.ds(off * 128, CHUNK)])
            return carry

        lax.fori_loop(0, n_iters, body, 0)

    return gather_kernel


@jax.jit
def kernel(ids, pe):
    b, h = ids.shape
    B = b * h
    ids2 = ids.reshape(B // 128, 128)
    out = _make_kernel(B)(ids2, pe)
    return out.reshape(b, h, D)


# speedup vs baseline: 10.3971x; 1.1566x over previous
import functools

import jax
import jax.numpy as jnp
from jax import lax
from jax.experimental import pallas as pl
from jax.experimental.pallas import tpu as pltpu
from jax.experimental.pallas import tpu_sc as plsc

D = 128
NC = 2
NS = 16
NW = NC * NS
NBUF = 4


@functools.lru_cache(maxsize=None)
def _make_kernel(B):
    n = B // (NW * 128)
    assert B % (NW * 128) == 0 and n % NBUF == 0 and n // NBUF >= 2
    n_outer = n // NBUF

    mesh = plsc.VectorSubcoreMesh(core_axis_name="c", subcore_axis_name="s")

    @functools.partial(
        pl.kernel,
        out_type=jax.ShapeDtypeStruct((B, D), jnp.float32),
        mesh=mesh,
        scratch_types=(
            [pltpu.VMEM((1, 128), jnp.int32) for _ in range(NBUF)]
            + [pltpu.VMEM((128, D), jnp.float32) for _ in range(NBUF)]
            + [pltpu.SemaphoreType.DMA for _ in range(3 * NBUF)]
        ),
    )
    def gather_kernel(ids_hbm, pe_hbm, out_hbm, *scratch):
        idx = scratch[0:NBUF]
        rows = scratch[NBUF:2 * NBUF]
        isem = scratch[2 * NBUF:3 * NBUF]
        gsem = scratch[3 * NBUF:4 * NBUF]
        osem = scratch[4 * NBUF:5 * NBUF]

        wid = lax.axis_index("s") * NC + lax.axis_index("c")
        row_base = wid * n

        def fl(g, b):
            pltpu.make_async_copy(
                ids_hbm.at[pl.ds(row_base + g, 1)], idx[b], isem[b]).start()

        def wi(b):
            pltpu.make_async_copy(
                ids_hbm.at[pl.ds(0, 1)], idx[b], isem[b]).wait()

        def fg(b):
            pltpu.make_async_copy(
                pe_hbm.at[idx[b].at[0]], rows[b], gsem[b]).start()

        def wg(b):
            pltpu.make_async_copy(
                pe_hbm.at[pl.ds(0, 128)], rows[b], gsem[b]).wait()

        def fs(g, b):
            pltpu.make_async_copy(
                rows[b], out_hbm.at[pl.ds((row_base + g) * 128, 128)],
                osem[b]).start()

        def ws(b):
            pltpu.make_async_copy(
                rows[b], out_hbm.at[pl.ds(0, 128)], osem[b]).wait()

        for b in range(NBUF):
            fl(b, b)
        wi(0); fg(0)
        wi(1); fg(1)

        wg(0); fs(0, 0); fl(4, 0); wi(2); fg(2)
        wg(1); fs(1, 1); fl(5, 1); wi(3); fg(3)
        wg(2); fs(2, 2); fl(6, 2); ws(0); wi(0); fg(0)
        wg(3); fs(3, 3); fl(7, 3); ws(1); wi(1); fg(1)

        def body(i, carry):
            base = i * NBUF
            for u in range(NBUF):
                b = u
                c = (u + 2) % NBUF
                g = base + u
                wg(b)
                fs(g, b)
                fl(g + NBUF, b)
                ws(c)
                wi(c)
                fg(c)
            return carry

        lax.fori_loop(1, n_outer - 1, body, 0)

        g0 = n - NBUF
        wg(0); fs(g0 + 0, 0); ws(2); wi(2); fg(2)
        wg(1); fs(g0 + 1, 1); ws(3); wi(3); fg(3)
        wg(2); fs(g0 + 2, 2); ws(0)
        wg(3); fs(g0 + 3, 3); ws(1)
        ws(2); ws(3)

    return gather_kernel


@jax.jit
def kernel(ids, pe):
    b, h = ids.shape
    B = b * h
    ids2 = ids.reshape(B // 128, 128)
    out = _make_kernel(B)(ids2, pe)
    return out.reshape(b, h, D)
